# Initial kernel scaffold; baseline (speedup 1.0000x reference)
import jax
import jax.numpy as jnp
from jax.experimental import pallas as pl


def kernel(token_ids, W):
    raise NotImplementedError("write your pallas kernel here")


# speedup vs baseline: 4.0869x; 4.0869x over previous
import functools

import jax
import jax.numpy as jnp
from jax import lax
from jax.experimental import pallas as pl
from jax.experimental.pallas import tpu as pltpu
from jax.experimental.pallas import tpu_sc as plsc

_NC = 2
_NS = 16
_NW = _NC * _NS
_BLK = 128


def kernel(token_ids, W):
    B, S = token_ids.shape
    n = B * S
    dim = W.shape[1]
    b_per_w = n // _NW
    blocks = b_per_w // _BLK
    idx = token_ids.reshape(n // _BLK, _BLK)

    mesh = plsc.VectorSubcoreMesh(core_axis_name="c", subcore_axis_name="s")

    @functools.partial(
        pl.kernel,
        mesh=mesh,
        out_type=jax.ShapeDtypeStruct((n, dim), W.dtype),
        scratch_types=[
            pltpu.VMEM((blocks, _BLK), jnp.int32),
            pltpu.VMEM((_BLK, dim), jnp.float32),
            pltpu.SemaphoreType.DMA,
        ],
        compiler_params=pltpu.CompilerParams(use_tc_tiling_on_sc=False),
    )
    def gather_kernel(w_hbm, i_hbm, o_hbm, idx_v, rows_v, sem):
        wid = lax.axis_index("s") * _NC + lax.axis_index("c")
        base = wid * b_per_w
        pltpu.sync_copy(i_hbm.at[pl.ds(wid * blocks, blocks)], idx_v)

        @pl.loop(0, blocks)
        def _(j):
            pltpu.async_copy(w_hbm.at[idx_v.at[j]], rows_v, sem).wait()
            pltpu.sync_copy(rows_v, o_hbm.at[pl.ds(base + j * _BLK, _BLK)])

    out = gather_kernel(W, idx)
    return out.reshape(B, S, dim)


# speedup vs baseline: 4.6003x; 1.1256x over previous
import functools

import jax
import jax.numpy as jnp
from jax import lax
from jax.experimental import pallas as pl
from jax.experimental.pallas import tpu as pltpu
from jax.experimental.pallas import tpu_sc as plsc

_NC = 2
_NS = 16
_NW = _NC * _NS
_BLK = 128
_SUB = 5
_SUPER = _BLK * _SUB


def kernel(token_ids, W):
    B, S = token_ids.shape
    n = B * S
    dim = W.shape[1]
    b_per_w = n // _NW
    blocks = b_per_w // _BLK
    rounds = b_per_w // _SUPER
    idx = token_ids.reshape(n // _BLK, _BLK)

    mesh = plsc.VectorSubcoreMesh(core_axis_name="c", subcore_axis_name="s")

    @functools.partial(
        pl.kernel,
        mesh=mesh,
        out_type=jax.ShapeDtypeStruct((n, dim), W.dtype),
        scratch_types=[
            pltpu.VMEM((blocks, _BLK), jnp.int32),
            pltpu.VMEM((_SUPER, dim), jnp.float32),
            pltpu.VMEM((_SUPER, dim), jnp.float32),
            pltpu.SemaphoreType.DMA,
            pltpu.SemaphoreType.DMA,
            pltpu.SemaphoreType.DMA,
            pltpu.SemaphoreType.DMA,
        ],
        compiler_params=pltpu.CompilerParams(use_tc_tiling_on_sc=False),
    )
    def gather_kernel(w_hbm, i_hbm, o_hbm, idx_v, buf0, buf1,
                      gsem0, gsem1, wsem0, wsem1):
        wid = lax.axis_index("s") * _NC + lax.axis_index("c")
        base = wid * b_per_w
        pltpu.sync_copy(i_hbm.at[pl.ds(wid * blocks, blocks)], idx_v)

        bufs = (buf0, buf1)
        gsems = (gsem0, gsem1)
        wsems = (wsem0, wsem1)

        def fire(r, slot):
            for b in range(_SUB):
                pltpu.async_copy(
                    w_hbm.at[idx_v.at[r * _SUB + b]],
                    bufs[slot].at[pl.ds(b * _BLK, _BLK)],
                    gsems[slot],
                )

        def drain_gathers(slot):
            pltpu.make_async_copy(
                w_hbm.at[pl.ds(0, _SUPER)], bufs[slot], gsems[slot]
            ).wait()

        def start_wb(r, slot):
            pltpu.async_copy(
                bufs[slot], o_hbm.at[pl.ds(base + r * _SUPER, _SUPER)],
                wsems[slot],
            )

        def drain_wb(slot):
            pltpu.make_async_copy(
                bufs[slot], o_hbm.at[pl.ds(base, _SUPER)], wsems[slot]
            ).wait()

        fire(0, 0)
        drain_gathers(0)
        fire(1, 1)
        start_wb(0, 0)
        drain_gathers(1)
        drain_wb(0)
        fire(2, 0)
        start_wb(1, 1)

        @pl.loop(2, rounds - 2, step=2)
        def _(j):
            drain_gathers(0)
            drain_wb(1)
            fire(j + 1, 1)
            start_wb(j, 0)
            drain_gathers(1)
            drain_wb(0)
            fire(j + 2, 0)
            start_wb(j + 1, 1)

        drain_gathers(0)
        drain_wb(1)
        fire(rounds - 1, 1)
        start_wb(rounds - 2, 0)
        drain_gathers(1)
        drain_wb(0)
        start_wb(rounds - 1, 1)
        drain_wb(1)

    out = gather_kernel(W, idx)
    return out.reshape(B, S, dim)
